# baseline (speedup 1.0000x reference)
import functools

import jax
import jax.numpy as jnp
from jax import lax
from jax.experimental import pallas as pl
from jax.experimental.pallas import tpu as pltpu
from jax.experimental.pallas import tpu_sc as plsc

N = 10000
D = 128
G = 64
NC = 2
NS = 16
CHUNK = 128
CH0 = 160
CH1 = 0
IB = 16
E_PAD = NS * (CH0 + CH1) * CHUNK
E0 = NS * CH0 * CHUNK
ROWS_PER_TILE = 624
TAIL_BASE = NS * ROWS_PER_TILE
ACC_ROWS = N + 32
ACC_TAIL = ACC_ROWS - TAIL_BASE

_MESH = plsc.VectorSubcoreMesh(core_axis_name="c", subcore_axis_name="s")


@functools.partial(
    pl.kernel,
    out_type=jax.ShapeDtypeStruct((NC, N, D), jnp.float32),
    mesh=_MESH,
    scratch_types=[
        pltpu.VMEM_SHARED((ACC_ROWS, D), jnp.float32),
        pltpu.VMEM((IB, CHUNK), jnp.int32),
        pltpu.VMEM((IB, CHUNK), jnp.int32),
        pltpu.VMEM((CHUNK, D), jnp.float32),
        pltpu.VMEM((CHUNK, D), jnp.float32),
        pltpu.SemaphoreType.DMA,
        pltpu.SemaphoreType.DMA,
    ],
)
def _mp_sc(h_hbm, row0_hbm, col0_hbm, out_hbm, acc,
           col_idx, row_idx, buf_a, buf_b, sem_a, sem_b):
    c = lax.axis_index("c")
    s = lax.axis_index("s")
    base = s * ROWS_PER_TILE

    z = jnp.zeros((16,), jnp.float32)

    def zrow(r, carry):
        def zcol(q, carry2):
            buf_a[r, pl.ds(q * 16, 16)] = z
            return carry2
        lax.fori_loop(0, D // 16, zcol, 0)
        return carry

    lax.fori_loop(0, CHUNK, zrow, 0)

    @pl.when(c == 0)
    def _zero_acc():
        for k in range(ROWS_PER_TILE // CHUNK):
            pltpu.sync_copy(buf_a, acc.at[pl.ds(base + k * CHUNK, CHUNK)])
        rem = ROWS_PER_TILE % CHUNK
        pltpu.sync_copy(buf_a.at[pl.ds(0, rem)],
                        acc.at[pl.ds(base + ROWS_PER_TILE - rem, rem)])

        @pl.when(s == 0)
        def _zero_tail():
            pltpu.sync_copy(buf_a.at[pl.ds(0, ACC_TAIL)],
                            acc.at[pl.ds(TAIL_BASE, ACC_TAIL)])

    plsc.subcore_barrier()

    def run(rows_hbm, cols_hbm, nchunks):
        def group(g, gcarry):
            pltpu.sync_copy(cols_hbm.at[s, pl.ds(g * IB, IB)], col_idx)
            pltpu.sync_copy(rows_hbm.at[s, pl.ds(g * IB, IB)], row_idx)

            def pair(j, carry):
                k0 = 2 * j
                k1 = 2 * j + 1
                cp_a = pltpu.async_copy(h_hbm.at[col_idx.at[k0]], buf_a,
                                        sem_a)
                cp_b = pltpu.async_copy(h_hbm.at[col_idx.at[k1]], buf_b,
                                        sem_b)
                cp_a.wait()
                pltpu.sync_copy(buf_a, acc.at[row_idx.at[k0]], add=True)
                cp_b.wait()
                pltpu.sync_copy(buf_b, acc.at[row_idx.at[k1]], add=True)
                return carry

            lax.fori_loop(0, IB // 2, pair, 0)
            return gcarry

        lax.fori_loop(0, nchunks // IB, group, 0)

    @pl.when(c == 0)
    def _run0():
        run(row0_hbm, col0_hbm, CH0)

    plsc.subcore_barrier()

    @pl.when(c == 0)
    def _writeout():
        pltpu.sync_copy(acc.at[pl.ds(base, ROWS_PER_TILE)],
                        out_hbm.at[0, pl.ds(base, ROWS_PER_TILE)])

        @pl.when(s == 0)
        def _out_tail():
            pltpu.sync_copy(acc.at[pl.ds(TAIL_BASE, N - TAIL_BASE)],
                            out_hbm.at[0, pl.ds(TAIL_BASE, N - TAIL_BASE)])


_BLK = 1000


def _affine_body(a_ref, h_ref, w_ref, b_ref, o_ref):
    a = a_ref[0] + 2.0 * h_ref[...]
    o_ref[...] = jnp.maximum(
        jnp.dot(a, w_ref[...], preferred_element_type=jnp.float32)
        + b_ref[...], 0.0)


def _affine_relu(aggr, h, w, b):
    return pl.pallas_call(
        _affine_body,
        grid=(N // _BLK,),
        in_specs=[
            pl.BlockSpec((NC, _BLK, D), lambda i: (0, i, 0)),
            pl.BlockSpec((_BLK, D), lambda i: (i, 0)),
            pl.BlockSpec((D, D), lambda i: (0, 0)),
            pl.BlockSpec((1, D), lambda i: (0, 0)),
        ],
        out_specs=pl.BlockSpec((_BLK, D), lambda i: (i, 0)),
        out_shape=jax.ShapeDtypeStruct((N, D), jnp.float32),
    )(aggr, h, w, b.reshape(1, D))


def _pool_body(a_ref, h_ref, w_ref, b_ref, batch_ref, wout_ref, bout_ref,
               o_ref, sums_ref, counts_ref):
    i = pl.program_id(0)

    @pl.when(i == 0)
    def _init():
        sums_ref[...] = jnp.zeros_like(sums_ref)
        counts_ref[...] = jnp.zeros_like(counts_ref)

    a = a_ref[0] + 2.0 * h_ref[...]
    h = jnp.maximum(
        jnp.dot(a, w_ref[...], preferred_element_type=jnp.float32)
        + b_ref[...], 0.0)
    b = batch_ref[0]
    onehot = (b.reshape(_BLK, 1)
              == lax.broadcasted_iota(jnp.int32, (_BLK, G), 1)
              ).astype(jnp.float32)
    sums_ref[...] += lax.dot_general(
        onehot, h, (((0,), (0,)), ((), ())),
        preferred_element_type=jnp.float32)
    counts_ref[...] += jnp.sum(onehot, axis=0, keepdims=True)

    @pl.when(i == (N // _BLK) - 1)
    def _final():
        pooled = sums_ref[...] / jnp.maximum(counts_ref[...], 1.0).reshape(G, 1)
        o_ref[...] = (jnp.dot(pooled, wout_ref[...],
                              preferred_element_type=jnp.float32)
                      + bout_ref[...])


def _pool_project(aggr, h, w, b, batch3d, wout, bout):
    return pl.pallas_call(
        _pool_body,
        grid=(N // _BLK,),
        in_specs=[
            pl.BlockSpec((NC, _BLK, D), lambda i: (0, i, 0)),
            pl.BlockSpec((_BLK, D), lambda i: (i, 0)),
            pl.BlockSpec((D, D), lambda i: (0, 0)),
            pl.BlockSpec((1, D), lambda i: (0, 0)),
            pl.BlockSpec((1, 1, _BLK), lambda i: (i, 0, 0)),
            pl.BlockSpec((D, D), lambda i: (0, 0)),
            pl.BlockSpec((1, D), lambda i: (0, 0)),
        ],
        out_specs=pl.BlockSpec((G, D), lambda i: (0, 0)),
        out_shape=jax.ShapeDtypeStruct((G, D), jnp.float32),
        scratch_shapes=[
            pltpu.VMEM((G, D), jnp.float32),
            pltpu.VMEM((1, G), jnp.float32),
        ],
    )(aggr, h, w, b.reshape(1, D), batch3d, wout, bout.reshape(1, D))


def kernel(x, edge_index, batch, W1, b1, W2, b2, Wout, bout):
    e = edge_index.shape[1]
    pad = E_PAD - e
    row = jnp.concatenate([edge_index[0], jnp.full((pad,), N, jnp.int32)])
    col = jnp.concatenate([edge_index[1], jnp.zeros((pad,), jnp.int32)])
    row0 = row.reshape(NS, CH0, CHUNK)
    col0 = col.reshape(NS, CH0, CHUNK)
    batch3d = batch.reshape(N // _BLK, 1, _BLK)

    aggr1 = _mp_sc(x, row0, col0)
    h1 = _affine_relu(aggr1, x, W1, b1)
    aggr2 = _mp_sc(h1, row0, col0)
    return _pool_project(aggr2, h1, W2, b2, batch3d, Wout, bout)


# speedup vs baseline: 1.0793x; 1.0793x over previous
import functools

import jax
import jax.numpy as jnp
from jax import lax
from jax.experimental import pallas as pl
from jax.experimental.pallas import tpu as pltpu
from jax.experimental.pallas import tpu_sc as plsc

N = 10000
D = 128
G = 64
NC = 2
NS = 16
CHUNK = 128
CH0 = 80
CH1 = 80
IB = 16
E_PAD = NS * (CH0 + CH1) * CHUNK
E0 = NS * CH0 * CHUNK
ROWS_PER_TILE = 624
TAIL_BASE = NS * ROWS_PER_TILE
TAIL = N - TAIL_BASE
ACC_ROWS = N + CHUNK

_MESH = plsc.VectorSubcoreMesh(core_axis_name="c", subcore_axis_name="s")


@functools.partial(
    pl.kernel,
    out_type=jax.ShapeDtypeStruct((NC, N, D), jnp.float32),
    mesh=_MESH,
    scratch_types=[
        pltpu.VMEM_SHARED((ACC_ROWS, D), jnp.float32),
        pltpu.VMEM((IB, CHUNK), jnp.int32),
        pltpu.VMEM((IB, CHUNK), jnp.int32),
        pltpu.VMEM((CHUNK, D), jnp.float32),
        pltpu.VMEM((CHUNK, D), jnp.float32),
        pltpu.SemaphoreType.DMA,
        pltpu.SemaphoreType.DMA,
    ],
)
def _mp_sc(h_hbm, row0_hbm, col0_hbm, row1_hbm, col1_hbm, out_hbm, acc,
           col_idx, row_idx, buf_a, buf_b, sem_a, sem_b):
    c = lax.axis_index("c")
    s = lax.axis_index("s")
    base = s * ROWS_PER_TILE

    z = jnp.zeros((16,), jnp.float32)

    def zrow(r, carry):
        def zcol(q, carry2):
            buf_a[r, pl.ds(q * 16, 16)] = z
            return carry2
        lax.fori_loop(0, D // 16, zcol, 0)
        return carry

    lax.fori_loop(0, CHUNK, zrow, 0)
    for k in range(ROWS_PER_TILE // CHUNK):
        pltpu.sync_copy(buf_a, acc.at[pl.ds(base + k * CHUNK, CHUNK)])
    rem = ROWS_PER_TILE % CHUNK
    pltpu.sync_copy(buf_a.at[pl.ds(0, rem)],
                    acc.at[pl.ds(base + ROWS_PER_TILE - rem, rem)])

    @pl.when(s == 0)
    def _zero_tail():
        pltpu.sync_copy(buf_a.at[pl.ds(0, TAIL)],
                        acc.at[pl.ds(TAIL_BASE, TAIL)])

    plsc.subcore_barrier()

    def run(rows_hbm, cols_hbm, nchunks):
        def group(g, gcarry):
            pltpu.sync_copy(cols_hbm.at[s, pl.ds(g * IB, IB)], col_idx)
            pltpu.sync_copy(rows_hbm.at[s, pl.ds(g * IB, IB)], row_idx)

            def pair(j, carry):
                k0 = 2 * j
                k1 = 2 * j + 1
                cp_a = pltpu.async_copy(h_hbm.at[col_idx.at[k0]], buf_a,
                                        sem_a)
                cp_b = pltpu.async_copy(h_hbm.at[col_idx.at[k1]], buf_b,
                                        sem_b)
                cp_a.wait()
                pltpu.sync_copy(buf_a, acc.at[row_idx.at[k0]], add=True)
                cp_b.wait()
                pltpu.sync_copy(buf_b, acc.at[row_idx.at[k1]], add=True)
                return carry

            lax.fori_loop(0, IB // 2, pair, 0)
            return gcarry

        lax.fori_loop(0, nchunks // IB, group, 0)

    @pl.when(c == 0)
    def _run0():
        run(row0_hbm, col0_hbm, CH0)

    @pl.when(c == 1)
    def _run1():
        run(row1_hbm, col1_hbm, CH1)

    plsc.subcore_barrier()

    pltpu.sync_copy(acc.at[pl.ds(base, ROWS_PER_TILE)],
                    out_hbm.at[c, pl.ds(base, ROWS_PER_TILE)])

    @pl.when(s == 0)
    def _out_tail():
        pltpu.sync_copy(acc.at[pl.ds(TAIL_BASE, TAIL)],
                        out_hbm.at[c, pl.ds(TAIL_BASE, TAIL)])


_BLK = 1000


def _affine_body(a_ref, h_ref, w_ref, b_ref, o_ref):
    a = a_ref[0] + a_ref[1] + 2.0 * h_ref[...]
    o_ref[...] = jnp.maximum(
        jnp.dot(a, w_ref[...], preferred_element_type=jnp.float32)
        + b_ref[...], 0.0)


def _affine_relu(aggr, h, w, b):
    return pl.pallas_call(
        _affine_body,
        grid=(N // _BLK,),
        in_specs=[
            pl.BlockSpec((NC, _BLK, D), lambda i: (0, i, 0)),
            pl.BlockSpec((_BLK, D), lambda i: (i, 0)),
            pl.BlockSpec((D, D), lambda i: (0, 0)),
            pl.BlockSpec((1, D), lambda i: (0, 0)),
        ],
        out_specs=pl.BlockSpec((_BLK, D), lambda i: (i, 0)),
        out_shape=jax.ShapeDtypeStruct((N, D), jnp.float32),
    )(aggr, h, w, b.reshape(1, D))


def _pool_body(a_ref, h_ref, w_ref, b_ref, batch_ref, wout_ref, bout_ref,
               o_ref, sums_ref, counts_ref):
    i = pl.program_id(0)

    @pl.when(i == 0)
    def _init():
        sums_ref[...] = jnp.zeros_like(sums_ref)
        counts_ref[...] = jnp.zeros_like(counts_ref)

    a = a_ref[0] + a_ref[1] + 2.0 * h_ref[...]
    h = jnp.maximum(
        jnp.dot(a, w_ref[...], preferred_element_type=jnp.float32)
        + b_ref[...], 0.0)
    b = batch_ref[0]
    onehot = (b.reshape(_BLK, 1)
              == lax.broadcasted_iota(jnp.int32, (_BLK, G), 1)
              ).astype(jnp.float32)
    sums_ref[...] += lax.dot_general(
        onehot, h, (((0,), (0,)), ((), ())),
        preferred_element_type=jnp.float32)
    counts_ref[...] += jnp.sum(onehot, axis=0, keepdims=True)

    @pl.when(i == (N // _BLK) - 1)
    def _final():
        pooled = sums_ref[...] / jnp.maximum(counts_ref[...], 1.0).reshape(G, 1)
        o_ref[...] = (jnp.dot(pooled, wout_ref[...],
                              preferred_element_type=jnp.float32)
                      + bout_ref[...])


def _pool_project(aggr, h, w, b, batch3d, wout, bout):
    return pl.pallas_call(
        _pool_body,
        grid=(N // _BLK,),
        in_specs=[
            pl.BlockSpec((NC, _BLK, D), lambda i: (0, i, 0)),
            pl.BlockSpec((_BLK, D), lambda i: (i, 0)),
            pl.BlockSpec((D, D), lambda i: (0, 0)),
            pl.BlockSpec((1, D), lambda i: (0, 0)),
            pl.BlockSpec((1, 1, _BLK), lambda i: (i, 0, 0)),
            pl.BlockSpec((D, D), lambda i: (0, 0)),
            pl.BlockSpec((1, D), lambda i: (0, 0)),
        ],
        out_specs=pl.BlockSpec((G, D), lambda i: (0, 0)),
        out_shape=jax.ShapeDtypeStruct((G, D), jnp.float32),
        scratch_shapes=[
            pltpu.VMEM((G, D), jnp.float32),
            pltpu.VMEM((1, G), jnp.float32),
        ],
    )(aggr, h, w, b.reshape(1, D), batch3d, wout, bout.reshape(1, D))


def kernel(x, edge_index, batch, W1, b1, W2, b2, Wout, bout):
    e = edge_index.shape[1]
    pad = E_PAD - e
    dummy = N + (jnp.arange(pad, dtype=jnp.int32) % CHUNK)
    row = jnp.concatenate([edge_index[0], dummy])
    col = jnp.concatenate([edge_index[1], jnp.zeros((pad,), jnp.int32)])
    row0 = row[:E0].reshape(NS, CH0, CHUNK)
    col0 = col[:E0].reshape(NS, CH0, CHUNK)
    row1 = row[E0:].reshape(NS, CH1, CHUNK)
    col1 = col[E0:].reshape(NS, CH1, CHUNK)
    batch3d = batch.reshape(N // _BLK, 1, _BLK)

    aggr1 = _mp_sc(x, row0, col0, row1, col1)
    h1 = _affine_relu(aggr1, x, W1, b1)
    aggr2 = _mp_sc(h1, row0, col0, row1, col1)
    return _pool_project(aggr2, h1, W2, b2, batch3d, Wout, bout)


# speedup vs baseline: 3.4769x; 3.2215x over previous
import functools

import jax
import jax.numpy as jnp
from jax import lax
from jax.experimental import pallas as pl
from jax.experimental.pallas import tpu as pltpu
from jax.experimental.pallas import tpu_sc as plsc

N = 10000
D = 128
G = 64
NC = 2
NS = 16
CHUNK = 128
CH0 = 80
CH1 = 80
IB = 16
E_PAD = NS * (CH0 + CH1) * CHUNK
E0 = NS * CH0 * CHUNK
ROWS_PER_TILE = 624
TAIL_BASE = NS * ROWS_PER_TILE
TAIL = N - TAIL_BASE
ACC_ROWS = N + CHUNK

_MESH = plsc.VectorSubcoreMesh(core_axis_name="c", subcore_axis_name="s")


@functools.partial(
    pl.kernel,
    out_type=jax.ShapeDtypeStruct((NC, N, D), jnp.float32),
    mesh=_MESH,
    scratch_types=[
        pltpu.VMEM_SHARED((ACC_ROWS, D), jnp.float32),
        pltpu.VMEM((IB, CHUNK), jnp.int32),
        pltpu.VMEM((IB, CHUNK), jnp.int32),
        pltpu.VMEM((CHUNK, D), jnp.float32),
        pltpu.VMEM((CHUNK, D), jnp.float32),
        pltpu.SemaphoreType.DMA,
        pltpu.SemaphoreType.DMA,
    ],
)
def _mp_sc(h_hbm, row0_hbm, col0_hbm, row1_hbm, col1_hbm, out_hbm, acc,
           col_idx, row_idx, buf_a, buf_b, sem_a, sem_b):
    c = lax.axis_index("c")
    s = lax.axis_index("s")
    base = s * ROWS_PER_TILE

    z = jnp.zeros((16,), jnp.float32)

    def zrow(r, carry):
        def zcol(q, carry2):
            buf_a[r, pl.ds(q * 16, 16)] = z
            return carry2
        lax.fori_loop(0, D // 16, zcol, 0)
        return carry

    lax.fori_loop(0, CHUNK, zrow, 0)
    for k in range(ROWS_PER_TILE // CHUNK):
        pltpu.sync_copy(buf_a, acc.at[pl.ds(base + k * CHUNK, CHUNK)])
    rem = ROWS_PER_TILE % CHUNK
    pltpu.sync_copy(buf_a.at[pl.ds(0, rem)],
                    acc.at[pl.ds(base + ROWS_PER_TILE - rem, rem)])

    @pl.when(s == 0)
    def _zero_tail():
        pltpu.sync_copy(buf_a.at[pl.ds(0, TAIL)],
                        acc.at[pl.ds(TAIL_BASE, TAIL)])

    plsc.subcore_barrier()

    def run(rows_hbm, cols_hbm, nchunks):
        def group(g, gcarry):
            pltpu.sync_copy(cols_hbm.at[s, pl.ds(g * IB, IB)], col_idx)
            pltpu.sync_copy(rows_hbm.at[s, pl.ds(g * IB, IB)], row_idx)

            def pair(j, carry):
                k0 = 2 * j
                k1 = 2 * j + 1
                cp_a = pltpu.async_copy(h_hbm.at[col_idx.at[k0]], buf_a,
                                        sem_a)
                cp_b = pltpu.async_copy(h_hbm.at[col_idx.at[k1]], buf_b,
                                        sem_b)
                cp_a.wait()
                pltpu.sync_copy(buf_a, acc.at[row_idx.at[k0]], add=True)
                cp_b.wait()
                pltpu.sync_copy(buf_b, acc.at[row_idx.at[k1]], add=True)
                return carry

            lax.fori_loop(0, IB // 2, pair, 0)
            return gcarry

        lax.fori_loop(0, nchunks // IB, group, 0)

    @pl.when(c == 0)
    def _run0():
        run(row0_hbm, col0_hbm, CH0)

    @pl.when(c == 1)
    def _run1():
        run(row1_hbm, col1_hbm, CH1)

    plsc.subcore_barrier()

    pltpu.sync_copy(acc.at[pl.ds(base, ROWS_PER_TILE)],
                    out_hbm.at[c, pl.ds(base, ROWS_PER_TILE)])

    @pl.when(s == 0)
    def _out_tail():
        pltpu.sync_copy(acc.at[pl.ds(TAIL_BASE, TAIL)],
                        out_hbm.at[c, pl.ds(TAIL_BASE, TAIL)])


_BLK = 1000


def _affine_body(a_ref, h_ref, w_ref, b_ref, o_ref):
    a = a_ref[0] + a_ref[1] + 2.0 * h_ref[...]
    o_ref[...] = jnp.maximum(
        jnp.dot(a, w_ref[...], preferred_element_type=jnp.float32)
        + b_ref[...], 0.0)


def _affine_relu(aggr, h, w, b):
    return pl.pallas_call(
        _affine_body,
        grid=(N // _BLK,),
        in_specs=[
            pl.BlockSpec((NC, _BLK, D), lambda i: (0, i, 0)),
            pl.BlockSpec((_BLK, D), lambda i: (i, 0)),
            pl.BlockSpec((D, D), lambda i: (0, 0)),
            pl.BlockSpec((1, D), lambda i: (0, 0)),
        ],
        out_specs=pl.BlockSpec((_BLK, D), lambda i: (i, 0)),
        out_shape=jax.ShapeDtypeStruct((N, D), jnp.float32),
    )(aggr, h, w, b.reshape(1, D))


def _pool_body(a_ref, h_ref, w_ref, b_ref, batch_ref, wout_ref, bout_ref,
               o_ref, sums_ref, counts_ref):
    i = pl.program_id(0)

    @pl.when(i == 0)
    def _init():
        sums_ref[...] = jnp.zeros_like(sums_ref)
        counts_ref[...] = jnp.zeros_like(counts_ref)

    a = a_ref[0] + a_ref[1] + 2.0 * h_ref[...]
    h = jnp.maximum(
        jnp.dot(a, w_ref[...], preferred_element_type=jnp.float32)
        + b_ref[...], 0.0)
    b = batch_ref[0]
    onehot = (b.reshape(_BLK, 1)
              == lax.broadcasted_iota(jnp.int32, (_BLK, G), 1)
              ).astype(jnp.float32)
    sums_ref[...] += lax.dot_general(
        onehot, h, (((0,), (0,)), ((), ())),
        preferred_element_type=jnp.float32)
    counts_ref[...] += jnp.sum(onehot, axis=0, keepdims=True)

    @pl.when(i == (N // _BLK) - 1)
    def _final():
        pooled = sums_ref[...] / jnp.maximum(counts_ref[...], 1.0).reshape(G, 1)
        o_ref[...] = (jnp.dot(pooled, wout_ref[...],
                              preferred_element_type=jnp.float32)
                      + bout_ref[...])


def _pool_project(aggr, h, w, b, batch3d, wout, bout):
    return pl.pallas_call(
        _pool_body,
        grid=(N // _BLK,),
        in_specs=[
            pl.BlockSpec((NC, _BLK, D), lambda i: (0, i, 0)),
            pl.BlockSpec((_BLK, D), lambda i: (i, 0)),
            pl.BlockSpec((D, D), lambda i: (0, 0)),
            pl.BlockSpec((1, D), lambda i: (0, 0)),
            pl.BlockSpec((1, 1, _BLK), lambda i: (i, 0, 0)),
            pl.BlockSpec((D, D), lambda i: (0, 0)),
            pl.BlockSpec((1, D), lambda i: (0, 0)),
        ],
        out_specs=pl.BlockSpec((G, D), lambda i: (0, 0)),
        out_shape=jax.ShapeDtypeStruct((G, D), jnp.float32),
        scratch_shapes=[
            pltpu.VMEM((G, D), jnp.float32),
            pltpu.VMEM((1, G), jnp.float32),
        ],
    )(aggr, h, w, b.reshape(1, D), batch3d, wout, bout.reshape(1, D))


def kernel(x, edge_index, batch, W1, b1, W2, b2, Wout, bout):
    e = edge_index.shape[1]
    pad = E_PAD - e
    spread = jnp.arange(pad, dtype=jnp.int32)
    row = jnp.concatenate([edge_index[0], N + (spread % CHUNK)])
    col = jnp.concatenate([edge_index[1], spread % N])
    row0 = row[:E0].reshape(NS, CH0, CHUNK)
    col0 = col[:E0].reshape(NS, CH0, CHUNK)
    row1 = row[E0:].reshape(NS, CH1, CHUNK)
    col1 = col[E0:].reshape(NS, CH1, CHUNK)
    batch3d = batch.reshape(N // _BLK, 1, _BLK)

    aggr1 = _mp_sc(x, row0, col0, row1, col1)
    h1 = _affine_relu(aggr1, x, W1, b1)
    aggr2 = _mp_sc(h1, row0, col0, row1, col1)
    return _pool_project(aggr2, h1, W2, b2, batch3d, Wout, bout)


# speedup vs baseline: 3.6520x; 1.0504x over previous
import functools

import jax
import jax.numpy as jnp
from jax import lax
from jax.experimental import pallas as pl
from jax.experimental.pallas import tpu as pltpu
from jax.experimental.pallas import tpu_sc as plsc

N = 10000
D = 128
G = 64
NC = 2
NS = 16
CHUNK = 128
CH0 = 80
CH1 = 80
IB = 16
E_PAD = NS * (CH0 + CH1) * CHUNK
E0 = NS * CH0 * CHUNK
ROWS_PER_TILE = 624
TAIL_BASE = NS * ROWS_PER_TILE
TAIL = N - TAIL_BASE
ACC_ROWS = N + CHUNK

_MESH = plsc.VectorSubcoreMesh(core_axis_name="c", subcore_axis_name="s")


@functools.partial(
    pl.kernel,
    out_type=jax.ShapeDtypeStruct((NC, N, D), jnp.float32),
    mesh=_MESH,
    scratch_types=[
        pltpu.VMEM_SHARED((ACC_ROWS, D), jnp.float32),
        pltpu.VMEM((2, IB, CHUNK), jnp.int32),
        pltpu.VMEM((2, IB, CHUNK), jnp.int32),
        pltpu.VMEM((CHUNK, D), jnp.float32),
        pltpu.VMEM((CHUNK, D), jnp.float32),
        pltpu.SemaphoreType.DMA,
        pltpu.SemaphoreType.DMA,
        pltpu.SemaphoreType.DMA,
        pltpu.SemaphoreType.DMA,
    ],
)
def _mp_sc(h_hbm, row0_hbm, col0_hbm, row1_hbm, col1_hbm, out_hbm, acc,
           col_idx, row_idx, buf_a, buf_b, sem_a, sem_b, sem_sa, sem_sb):
    c = lax.axis_index("c")
    s = lax.axis_index("s")
    base = s * ROWS_PER_TILE

    z = jnp.zeros((16,), jnp.float32)

    def zrow(r, carry):
        for q in range(D // 16):
            buf_a[r, pl.ds(q * 16, 16)] = z
        return carry

    lax.fori_loop(0, CHUNK, zrow, 0)
    for k in range(ROWS_PER_TILE // CHUNK):
        pltpu.sync_copy(buf_a, acc.at[pl.ds(base + k * CHUNK, CHUNK)])
    rem = ROWS_PER_TILE % CHUNK
    pltpu.sync_copy(buf_a.at[pl.ds(0, rem)],
                    acc.at[pl.ds(base + ROWS_PER_TILE - rem, rem)])

    @pl.when(s == 0)
    def _zero_tail():
        pltpu.sync_copy(buf_a.at[pl.ds(0, TAIL)],
                        acc.at[pl.ds(TAIL_BASE, TAIL)])

    plsc.subcore_barrier()

    def run(rows_hbm, cols_hbm, nchunks):
        def group(g, gcarry):
            p = lax.rem(g, 2)
            pltpu.sync_copy(cols_hbm.at[s, pl.ds(g * IB, IB)], col_idx.at[p])
            pltpu.sync_copy(rows_hbm.at[s, pl.ds(g * IB, IB)], row_idx.at[p])

            def pair(j, carry):
                k0 = 2 * j
                k1 = 2 * j + 1
                not_first = jnp.logical_or(g > 0, j > 0)

                @pl.when(not_first)
                def _wa():
                    pltpu.make_async_copy(
                        h_hbm.at[pl.ds(0, CHUNK)], buf_a, sem_sa).wait()

                cp_a = pltpu.async_copy(h_hbm.at[col_idx.at[p, k0]], buf_a,
                                        sem_a)

                @pl.when(not_first)
                def _wb():
                    pltpu.make_async_copy(
                        h_hbm.at[pl.ds(0, CHUNK)], buf_b, sem_sb).wait()

                cp_b = pltpu.async_copy(h_hbm.at[col_idx.at[p, k1]], buf_b,
                                        sem_b)
                cp_a.wait()
                pltpu.async_copy(buf_a, acc.at[row_idx.at[p, k0]], sem_sa,
                                 add=True)
                cp_b.wait()
                pltpu.async_copy(buf_b, acc.at[row_idx.at[p, k1]], sem_sb,
                                 add=True)
                return carry

            lax.fori_loop(0, IB // 2, pair, 0)
            return gcarry

        lax.fori_loop(0, nchunks // IB, group, 0)
        pltpu.make_async_copy(h_hbm.at[pl.ds(0, CHUNK)], buf_a, sem_sa).wait()
        pltpu.make_async_copy(h_hbm.at[pl.ds(0, CHUNK)], buf_b, sem_sb).wait()

    @pl.when(c == 0)
    def _run0():
        run(row0_hbm, col0_hbm, CH0)

    @pl.when(c == 1)
    def _run1():
        run(row1_hbm, col1_hbm, CH1)

    plsc.subcore_barrier()

    pltpu.sync_copy(acc.at[pl.ds(base, ROWS_PER_TILE)],
                    out_hbm.at[c, pl.ds(base, ROWS_PER_TILE)])

    @pl.when(s == 0)
    def _out_tail():
        pltpu.sync_copy(acc.at[pl.ds(TAIL_BASE, TAIL)],
                        out_hbm.at[c, pl.ds(TAIL_BASE, TAIL)])


_BLK = 1000


def _affine_body(a_ref, h_ref, w_ref, b_ref, o_ref):
    a = a_ref[0] + a_ref[1] + 2.0 * h_ref[...]
    o_ref[...] = jnp.maximum(
        jnp.dot(a, w_ref[...], preferred_element_type=jnp.float32)
        + b_ref[...], 0.0)


def _affine_relu(aggr, h, w, b):
    return pl.pallas_call(
        _affine_body,
        grid=(N // _BLK,),
        in_specs=[
            pl.BlockSpec((NC, _BLK, D), lambda i: (0, i, 0)),
            pl.BlockSpec((_BLK, D), lambda i: (i, 0)),
            pl.BlockSpec((D, D), lambda i: (0, 0)),
            pl.BlockSpec((1, D), lambda i: (0, 0)),
        ],
        out_specs=pl.BlockSpec((_BLK, D), lambda i: (i, 0)),
        out_shape=jax.ShapeDtypeStruct((N, D), jnp.float32),
    )(aggr, h, w, b.reshape(1, D))


def _pool_body(a_ref, h_ref, w_ref, b_ref, batch_ref, wout_ref, bout_ref,
               o_ref, sums_ref, counts_ref):
    i = pl.program_id(0)

    @pl.when(i == 0)
    def _init():
        sums_ref[...] = jnp.zeros_like(sums_ref)
        counts_ref[...] = jnp.zeros_like(counts_ref)

    a = a_ref[0] + a_ref[1] + 2.0 * h_ref[...]
    h = jnp.maximum(
        jnp.dot(a, w_ref[...], preferred_element_type=jnp.float32)
        + b_ref[...], 0.0)
    b = batch_ref[0]
    onehot = (b.reshape(_BLK, 1)
              == lax.broadcasted_iota(jnp.int32, (_BLK, G), 1)
              ).astype(jnp.float32)
    sums_ref[...] += lax.dot_general(
        onehot, h, (((0,), (0,)), ((), ())),
        preferred_element_type=jnp.float32)
    counts_ref[...] += jnp.sum(onehot, axis=0, keepdims=True)

    @pl.when(i == (N // _BLK) - 1)
    def _final():
        pooled = sums_ref[...] / jnp.maximum(counts_ref[...], 1.0).reshape(G, 1)
        o_ref[...] = (jnp.dot(pooled, wout_ref[...],
                              preferred_element_type=jnp.float32)
                      + bout_ref[...])


def _pool_project(aggr, h, w, b, batch3d, wout, bout):
    return pl.pallas_call(
        _pool_body,
        grid=(N // _BLK,),
        in_specs=[
            pl.BlockSpec((NC, _BLK, D), lambda i: (0, i, 0)),
            pl.BlockSpec((_BLK, D), lambda i: (i, 0)),
            pl.BlockSpec((D, D), lambda i: (0, 0)),
            pl.BlockSpec((1, D), lambda i: (0, 0)),
            pl.BlockSpec((1, 1, _BLK), lambda i: (i, 0, 0)),
            pl.BlockSpec((D, D), lambda i: (0, 0)),
            pl.BlockSpec((1, D), lambda i: (0, 0)),
        ],
        out_specs=pl.BlockSpec((G, D), lambda i: (0, 0)),
        out_shape=jax.ShapeDtypeStruct((G, D), jnp.float32),
        scratch_shapes=[
            pltpu.VMEM((G, D), jnp.float32),
            pltpu.VMEM((1, G), jnp.float32),
        ],
    )(aggr, h, w, b.reshape(1, D), batch3d, wout, bout.reshape(1, D))


def kernel(x, edge_index, batch, W1, b1, W2, b2, Wout, bout):
    e = edge_index.shape[1]
    pad = E_PAD - e
    spread = jnp.arange(pad, dtype=jnp.int32)
    row = jnp.concatenate([edge_index[0], N + (spread % CHUNK)])
    col = jnp.concatenate([edge_index[1], spread % N])
    row0 = row[:E0].reshape(NS, CH0, CHUNK)
    col0 = col[:E0].reshape(NS, CH0, CHUNK)
    row1 = row[E0:].reshape(NS, CH1, CHUNK)
    col1 = col[E0:].reshape(NS, CH1, CHUNK)
    batch3d = batch.reshape(N // _BLK, 1, _BLK)

    aggr1 = _mp_sc(x, row0, col0, row1, col1)
    h1 = _affine_relu(aggr1, x, W1, b1)
    aggr2 = _mp_sc(h1, row0, col0, row1, col1)
    return _pool_project(aggr2, h1, W2, b2, batch3d, Wout, bout)


# speedup vs baseline: 3.7103x; 1.0160x over previous
import functools

import jax
import jax.numpy as jnp
from jax import lax
from jax.experimental import pallas as pl
from jax.experimental.pallas import tpu as pltpu
from jax.experimental.pallas import tpu_sc as plsc

N = 10000
D = 128
G = 64
NC = 2
NS = 16
CHUNK = 128
CH0 = 80
CH1 = 80
IB = 16
E_PAD = NS * (CH0 + CH1) * CHUNK
E0 = NS * CH0 * CHUNK
ROWS_PER_TILE = 624
TAIL_BASE = NS * ROWS_PER_TILE
TAIL = N - TAIL_BASE
ACC_ROWS = N + CHUNK

_MESH = plsc.VectorSubcoreMesh(core_axis_name="c", subcore_axis_name="s")


@functools.partial(
    pl.kernel,
    out_type=jax.ShapeDtypeStruct((NC, N, D), jnp.float32),
    mesh=_MESH,
    scratch_types=[
        pltpu.VMEM_SHARED((ACC_ROWS, D), jnp.float32),
        pltpu.VMEM((2, IB, CHUNK), jnp.int32),
        pltpu.VMEM((2, IB, CHUNK), jnp.int32),
        pltpu.VMEM((CHUNK, D), jnp.float32),
        pltpu.VMEM((CHUNK, D), jnp.float32),
        pltpu.SemaphoreType.DMA,
        pltpu.SemaphoreType.DMA,
        pltpu.SemaphoreType.DMA,
        pltpu.SemaphoreType.DMA,
        pltpu.SemaphoreType.DMA,
    ],
)
def _mp_sc(h_hbm, row0_hbm, col0_hbm, row1_hbm, col1_hbm, out_hbm, acc,
           col_idx, row_idx, buf_a, buf_b, sem_a, sem_b, sem_sa, sem_sb,
           sem_idx):
    c = lax.axis_index("c")
    s = lax.axis_index("s")
    base = s * ROWS_PER_TILE

    z = jnp.zeros((16,), jnp.float32)

    def zrow(r, carry):
        for q in range(D // 16):
            buf_a[r, pl.ds(q * 16, 16)] = z
        return carry

    lax.fori_loop(0, CHUNK, zrow, 0)
    for k in range(ROWS_PER_TILE // CHUNK):
        pltpu.sync_copy(buf_a, acc.at[pl.ds(base + k * CHUNK, CHUNK)])
    rem = ROWS_PER_TILE % CHUNK
    pltpu.sync_copy(buf_a.at[pl.ds(0, rem)],
                    acc.at[pl.ds(base + ROWS_PER_TILE - rem, rem)])

    @pl.when(s == 0)
    def _zero_tail():
        pltpu.sync_copy(buf_a.at[pl.ds(0, TAIL)],
                        acc.at[pl.ds(TAIL_BASE, TAIL)])

    plsc.subcore_barrier()

    def run(rows_hbm, cols_hbm, nchunks):
        ngroups = nchunks // IB
        pltpu.sync_copy(cols_hbm.at[s, pl.ds(0, IB)], col_idx.at[0])
        pltpu.sync_copy(rows_hbm.at[s, pl.ds(0, IB)], row_idx.at[0])

        def group(g, gcarry):
            p = lax.rem(g, 2)
            q = 1 - p

            @pl.when(g > 0)
            def _wait_idx():
                pltpu.make_async_copy(cols_hbm.at[s, pl.ds(0, IB)],
                                      col_idx.at[p], sem_idx).wait()
                pltpu.make_async_copy(rows_hbm.at[s, pl.ds(0, IB)],
                                      row_idx.at[p], sem_idx).wait()

            def pair(j, carry):
                @pl.when(jnp.logical_and(j == 1, g + 1 < ngroups))
                def _prefetch():
                    pltpu.async_copy(
                        cols_hbm.at[s, pl.ds((g + 1) * IB, IB)],
                        col_idx.at[q], sem_idx)
                    pltpu.async_copy(
                        rows_hbm.at[s, pl.ds((g + 1) * IB, IB)],
                        row_idx.at[q], sem_idx)
                k0 = 2 * j
                k1 = 2 * j + 1
                not_first = jnp.logical_or(g > 0, j > 0)

                @pl.when(not_first)
                def _wa():
                    pltpu.make_async_copy(
                        h_hbm.at[pl.ds(0, CHUNK)], buf_a, sem_sa).wait()

                cp_a = pltpu.async_copy(h_hbm.at[col_idx.at[p, k0]], buf_a,
                                        sem_a)

                @pl.when(not_first)
                def _wb():
                    pltpu.make_async_copy(
                        h_hbm.at[pl.ds(0, CHUNK)], buf_b, sem_sb).wait()

                cp_b = pltpu.async_copy(h_hbm.at[col_idx.at[p, k1]], buf_b,
                                        sem_b)
                cp_a.wait()
                pltpu.async_copy(buf_a, acc.at[row_idx.at[p, k0]], sem_sa,
                                 add=True)
                cp_b.wait()
                pltpu.async_copy(buf_b, acc.at[row_idx.at[p, k1]], sem_sb,
                                 add=True)
                return carry

            lax.fori_loop(0, IB // 2, pair, 0)
            return gcarry

        lax.fori_loop(0, ngroups, group, 0)
        pltpu.make_async_copy(h_hbm.at[pl.ds(0, CHUNK)], buf_a, sem_sa).wait()
        pltpu.make_async_copy(h_hbm.at[pl.ds(0, CHUNK)], buf_b, sem_sb).wait()

    @pl.when(c == 0)
    def _run0():
        run(row0_hbm, col0_hbm, CH0)

    @pl.when(c == 1)
    def _run1():
        run(row1_hbm, col1_hbm, CH1)

    plsc.subcore_barrier()

    pltpu.sync_copy(acc.at[pl.ds(base, ROWS_PER_TILE)],
                    out_hbm.at[c, pl.ds(base, ROWS_PER_TILE)])

    @pl.when(s == 0)
    def _out_tail():
        pltpu.sync_copy(acc.at[pl.ds(TAIL_BASE, TAIL)],
                        out_hbm.at[c, pl.ds(TAIL_BASE, TAIL)])


_BLK = 1000


def _affine_body(a_ref, h_ref, w_ref, b_ref, o_ref):
    a = a_ref[0] + a_ref[1] + 2.0 * h_ref[...]
    o_ref[...] = jnp.maximum(
        jnp.dot(a, w_ref[...], preferred_element_type=jnp.float32)
        + b_ref[...], 0.0)


def _affine_relu(aggr, h, w, b):
    return pl.pallas_call(
        _affine_body,
        grid=(N // _BLK,),
        in_specs=[
            pl.BlockSpec((NC, _BLK, D), lambda i: (0, i, 0)),
            pl.BlockSpec((_BLK, D), lambda i: (i, 0)),
            pl.BlockSpec((D, D), lambda i: (0, 0)),
            pl.BlockSpec((1, D), lambda i: (0, 0)),
        ],
        out_specs=pl.BlockSpec((_BLK, D), lambda i: (i, 0)),
        out_shape=jax.ShapeDtypeStruct((N, D), jnp.float32),
    )(aggr, h, w, b.reshape(1, D))


def _pool_body(a_ref, h_ref, w_ref, b_ref, batch_ref, wout_ref, bout_ref,
               o_ref, sums_ref, counts_ref):
    i = pl.program_id(0)

    @pl.when(i == 0)
    def _init():
        sums_ref[...] = jnp.zeros_like(sums_ref)
        counts_ref[...] = jnp.zeros_like(counts_ref)

    a = a_ref[0] + a_ref[1] + 2.0 * h_ref[...]
    h = jnp.maximum(
        jnp.dot(a, w_ref[...], preferred_element_type=jnp.float32)
        + b_ref[...], 0.0)
    b = batch_ref[0]
    onehot = (b.reshape(_BLK, 1)
              == lax.broadcasted_iota(jnp.int32, (_BLK, G), 1)
              ).astype(jnp.float32)
    sums_ref[...] += lax.dot_general(
        onehot, h, (((0,), (0,)), ((), ())),
        preferred_element_type=jnp.float32)
    counts_ref[...] += jnp.sum(onehot, axis=0, keepdims=True)

    @pl.when(i == (N // _BLK) - 1)
    def _final():
        pooled = sums_ref[...] / jnp.maximum(counts_ref[...], 1.0).reshape(G, 1)
        o_ref[...] = (jnp.dot(pooled, wout_ref[...],
                              preferred_element_type=jnp.float32)
                      + bout_ref[...])


def _pool_project(aggr, h, w, b, batch3d, wout, bout):
    return pl.pallas_call(
        _pool_body,
        grid=(N // _BLK,),
        in_specs=[
            pl.BlockSpec((NC, _BLK, D), lambda i: (0, i, 0)),
            pl.BlockSpec((_BLK, D), lambda i: (i, 0)),
            pl.BlockSpec((D, D), lambda i: (0, 0)),
            pl.BlockSpec((1, D), lambda i: (0, 0)),
            pl.BlockSpec((1, 1, _BLK), lambda i: (i, 0, 0)),
            pl.BlockSpec((D, D), lambda i: (0, 0)),
            pl.BlockSpec((1, D), lambda i: (0, 0)),
        ],
        out_specs=pl.BlockSpec((G, D), lambda i: (0, 0)),
        out_shape=jax.ShapeDtypeStruct((G, D), jnp.float32),
        scratch_shapes=[
            pltpu.VMEM((G, D), jnp.float32),
            pltpu.VMEM((1, G), jnp.float32),
        ],
    )(aggr, h, w, b.reshape(1, D), batch3d, wout, bout.reshape(1, D))


def kernel(x, edge_index, batch, W1, b1, W2, b2, Wout, bout):
    e = edge_index.shape[1]
    pad = E_PAD - e
    spread = jnp.arange(pad, dtype=jnp.int32)
    row = jnp.concatenate([edge_index[0], N + (spread & (CHUNK - 1))])
    col = jnp.concatenate([edge_index[1], spread])
    row0 = row[:E0].reshape(NS, CH0, CHUNK)
    col0 = col[:E0].reshape(NS, CH0, CHUNK)
    row1 = row[E0:].reshape(NS, CH1, CHUNK)
    col1 = col[E0:].reshape(NS, CH1, CHUNK)
    batch3d = batch.reshape(N // _BLK, 1, _BLK)

    aggr1 = _mp_sc(x, row0, col0, row1, col1)
    h1 = _affine_relu(aggr1, x, W1, b1)
    aggr2 = _mp_sc(h1, row0, col0, row1, col1)
    return _pool_project(aggr2, h1, W2, b2, batch3d, Wout, bout)


# speedup vs baseline: 3.8142x; 1.0280x over previous
import functools

import jax
import jax.numpy as jnp
from jax import lax
from jax.experimental import pallas as pl
from jax.experimental.pallas import tpu as pltpu
from jax.experimental.pallas import tpu_sc as plsc

N = 10000
D = 128
G = 64
NC = 2
NS = 16
CHUNK = 128
CH0 = 80
CH1 = 80
IB = 16
E_PAD = NS * (CH0 + CH1) * CHUNK
E0 = NS * CH0 * CHUNK
ROWS_PER_TILE = 624
TAIL_BASE = NS * ROWS_PER_TILE
TAIL = N - TAIL_BASE
ACC_ROWS = N + CHUNK

_MESH = plsc.VectorSubcoreMesh(core_axis_name="c", subcore_axis_name="s")


@functools.partial(
    pl.kernel,
    out_type=jax.ShapeDtypeStruct((NC, N, D), jnp.float32),
    mesh=_MESH,
    scratch_types=[
        pltpu.VMEM_SHARED((ACC_ROWS, D), jnp.float32),
        pltpu.VMEM((2, IB, CHUNK), jnp.int32),
        pltpu.VMEM((2, IB, CHUNK), jnp.int32),
        pltpu.VMEM((CHUNK, D), jnp.float32),
        pltpu.VMEM((CHUNK, D), jnp.float32),
        pltpu.SemaphoreType.DMA,
        pltpu.SemaphoreType.DMA,
        pltpu.SemaphoreType.DMA,
        pltpu.SemaphoreType.DMA,
        pltpu.SemaphoreType.DMA,
    ],
)
def _mp_sc(h_hbm, row0_hbm, col0_hbm, row1_hbm, col1_hbm, out_hbm, acc,
           col_idx, row_idx, buf_a, buf_b, sem_a, sem_b, sem_sa, sem_sb,
           sem_idx):
    c = lax.axis_index("c")
    s = lax.axis_index("s")
    base = s * ROWS_PER_TILE

    z = jnp.zeros((16,), jnp.float32)

    def zrow(r, carry):
        for q in range(D // 16):
            buf_a[r, pl.ds(q * 16, 16)] = z
        return carry

    lax.fori_loop(0, CHUNK, zrow, 0)
    for k in range(ROWS_PER_TILE // CHUNK):
        pltpu.sync_copy(buf_a, acc.at[pl.ds(base + k * CHUNK, CHUNK)])
    rem = ROWS_PER_TILE % CHUNK
    pltpu.sync_copy(buf_a.at[pl.ds(0, rem)],
                    acc.at[pl.ds(base + ROWS_PER_TILE - rem, rem)])

    @pl.when(s == 0)
    def _zero_tail():
        pltpu.sync_copy(buf_a.at[pl.ds(0, TAIL)],
                        acc.at[pl.ds(TAIL_BASE, TAIL)])

    plsc.subcore_barrier()

    def run(rows_hbm, cols_hbm, nchunks):
        ngroups = nchunks // IB
        pltpu.sync_copy(cols_hbm.at[s, pl.ds(0, IB)], col_idx.at[0])
        pltpu.sync_copy(rows_hbm.at[s, pl.ds(0, IB)], row_idx.at[0])

        def group(g, gcarry):
            p = lax.rem(g, 2)
            q = 1 - p

            @pl.when(g > 0)
            def _wait_idx():
                pltpu.make_async_copy(cols_hbm.at[s, pl.ds(0, IB)],
                                      col_idx.at[p], sem_idx).wait()
                pltpu.make_async_copy(rows_hbm.at[s, pl.ds(0, IB)],
                                      row_idx.at[p], sem_idx).wait()

            def pair(j, carry):
                @pl.when(jnp.logical_and(j == 1, g + 1 < ngroups))
                def _prefetch():
                    pltpu.async_copy(
                        cols_hbm.at[s, pl.ds((g + 1) * IB, IB)],
                        col_idx.at[q], sem_idx)
                    pltpu.async_copy(
                        rows_hbm.at[s, pl.ds((g + 1) * IB, IB)],
                        row_idx.at[q], sem_idx)
                k0 = 2 * j
                k1 = 2 * j + 1
                not_first = jnp.logical_or(g > 0, j > 0)

                @pl.when(not_first)
                def _wa():
                    pltpu.make_async_copy(
                        h_hbm.at[pl.ds(0, CHUNK)], buf_a, sem_sa).wait()

                cp_a = pltpu.async_copy(h_hbm.at[col_idx.at[p, k0]], buf_a,
                                        sem_a)

                @pl.when(not_first)
                def _wb():
                    pltpu.make_async_copy(
                        h_hbm.at[pl.ds(0, CHUNK)], buf_b, sem_sb).wait()

                cp_b = pltpu.async_copy(h_hbm.at[col_idx.at[p, k1]], buf_b,
                                        sem_b)
                cp_a.wait()
                pltpu.async_copy(buf_a, acc.at[row_idx.at[p, k0]], sem_sa,
                                 add=True)
                cp_b.wait()
                pltpu.async_copy(buf_b, acc.at[row_idx.at[p, k1]], sem_sb,
                                 add=True)
                return carry

            lax.fori_loop(0, IB // 2, pair, 0)
            return gcarry

        lax.fori_loop(0, ngroups, group, 0)
        pltpu.make_async_copy(h_hbm.at[pl.ds(0, CHUNK)], buf_a, sem_sa).wait()
        pltpu.make_async_copy(h_hbm.at[pl.ds(0, CHUNK)], buf_b, sem_sb).wait()

    @pl.when(c == 0)
    def _run0():
        run(row0_hbm, col0_hbm, CH0)

    @pl.when(c == 1)
    def _run1():
        run(row1_hbm, col1_hbm, CH1)

    plsc.subcore_barrier()

    pltpu.sync_copy(acc.at[pl.ds(base, ROWS_PER_TILE)],
                    out_hbm.at[c, pl.ds(base, ROWS_PER_TILE)])

    @pl.when(s == 0)
    def _out_tail():
        pltpu.sync_copy(acc.at[pl.ds(TAIL_BASE, TAIL)],
                        out_hbm.at[c, pl.ds(TAIL_BASE, TAIL)])


_BLK = 2000


def _affine_body(a_ref, h_ref, w_ref, b_ref, o_ref):
    a = a_ref[0] + a_ref[1] + 2.0 * h_ref[...]
    o_ref[...] = jnp.maximum(
        jnp.dot(a, w_ref[...], preferred_element_type=jnp.float32)
        + b_ref[...], 0.0)


def _affine_relu(aggr, h, w, b):
    return pl.pallas_call(
        _affine_body,
        grid=(N // _BLK,),
        in_specs=[
            pl.BlockSpec((NC, _BLK, D), lambda i: (0, i, 0)),
            pl.BlockSpec((_BLK, D), lambda i: (i, 0)),
            pl.BlockSpec((D, D), lambda i: (0, 0)),
            pl.BlockSpec((1, D), lambda i: (0, 0)),
        ],
        out_specs=pl.BlockSpec((_BLK, D), lambda i: (i, 0)),
        out_shape=jax.ShapeDtypeStruct((N, D), jnp.float32),
    )(aggr, h, w, b.reshape(1, D))


def _pool_body(a_ref, h_ref, w_ref, b_ref, batch_ref, wout_ref, bout_ref,
               o_ref, sums_ref, counts_ref):
    i = pl.program_id(0)

    @pl.when(i == 0)
    def _init():
        sums_ref[...] = jnp.zeros_like(sums_ref)
        counts_ref[...] = jnp.zeros_like(counts_ref)

    a = a_ref[0] + a_ref[1] + 2.0 * h_ref[...]
    h = jnp.maximum(
        jnp.dot(a, w_ref[...], preferred_element_type=jnp.float32)
        + b_ref[...], 0.0)
    b = batch_ref[0]
    onehot = (b.reshape(_BLK, 1)
              == lax.broadcasted_iota(jnp.int32, (_BLK, G), 1)
              ).astype(jnp.float32)
    sums_ref[...] += lax.dot_general(
        onehot, h, (((0,), (0,)), ((), ())),
        preferred_element_type=jnp.float32)
    counts_ref[...] += jnp.sum(onehot, axis=0, keepdims=True)

    @pl.when(i == (N // _BLK) - 1)
    def _final():
        pooled = sums_ref[...] / jnp.maximum(counts_ref[...], 1.0).reshape(G, 1)
        o_ref[...] = (jnp.dot(pooled, wout_ref[...],
                              preferred_element_type=jnp.float32)
                      + bout_ref[...])


def _pool_project(aggr, h, w, b, batch3d, wout, bout):
    return pl.pallas_call(
        _pool_body,
        grid=(N // _BLK,),
        in_specs=[
            pl.BlockSpec((NC, _BLK, D), lambda i: (0, i, 0)),
            pl.BlockSpec((_BLK, D), lambda i: (i, 0)),
            pl.BlockSpec((D, D), lambda i: (0, 0)),
            pl.BlockSpec((1, D), lambda i: (0, 0)),
            pl.BlockSpec((1, 1, _BLK), lambda i: (i, 0, 0)),
            pl.BlockSpec((D, D), lambda i: (0, 0)),
            pl.BlockSpec((1, D), lambda i: (0, 0)),
        ],
        out_specs=pl.BlockSpec((G, D), lambda i: (0, 0)),
        out_shape=jax.ShapeDtypeStruct((G, D), jnp.float32),
        scratch_shapes=[
            pltpu.VMEM((G, D), jnp.float32),
            pltpu.VMEM((1, G), jnp.float32),
        ],
    )(aggr, h, w, b.reshape(1, D), batch3d, wout, bout.reshape(1, D))


def kernel(x, edge_index, batch, W1, b1, W2, b2, Wout, bout):
    e = edge_index.shape[1]
    pad = E_PAD - e
    spread = jnp.arange(pad, dtype=jnp.int32)
    row0 = edge_index[0, :E0].reshape(NS, CH0, CHUNK)
    col0 = edge_index[1, :E0].reshape(NS, CH0, CHUNK)
    row1 = jnp.concatenate(
        [edge_index[0, E0:], N + (spread & (CHUNK - 1))]
    ).reshape(NS, CH1, CHUNK)
    col1 = jnp.concatenate(
        [edge_index[1, E0:], spread]).reshape(NS, CH1, CHUNK)
    batch3d = batch.reshape(N // _BLK, 1, _BLK)

    aggr1 = _mp_sc(x, row0, col0, row1, col1)
    h1 = _affine_relu(aggr1, x, W1, b1)
    aggr2 = _mp_sc(h1, row0, col0, row1, col1)
    return _pool_project(aggr2, h1, W2, b2, batch3d, Wout, bout)


# speedup vs baseline: 3.8359x; 1.0057x over previous
import functools

import jax
import jax.numpy as jnp
from jax import lax
from jax.experimental import pallas as pl
from jax.experimental.pallas import tpu as pltpu
from jax.experimental.pallas import tpu_sc as plsc

N = 10000
D = 128
G = 64
NC = 2
NS = 16
CHUNK = 128
CH0 = 80
CH1 = 80
IB = 16
E_PAD = NS * (CH0 + CH1) * CHUNK
E0 = NS * CH0 * CHUNK
ROWS_PER_TILE = 624
TAIL_BASE = NS * ROWS_PER_TILE
TAIL = N - TAIL_BASE
ACC_ROWS = N + CHUNK

_MESH = plsc.VectorSubcoreMesh(core_axis_name="c", subcore_axis_name="s")


@functools.partial(
    pl.kernel,
    out_type=jax.ShapeDtypeStruct((NC, N, D), jnp.float32),
    mesh=_MESH,
    scratch_types=[
        pltpu.VMEM_SHARED((ACC_ROWS, D), jnp.float32),
        pltpu.VMEM((2, IB, CHUNK), jnp.int32),
        pltpu.VMEM((2, IB, CHUNK), jnp.int32),
        pltpu.VMEM((CHUNK, D), jnp.float32),
        pltpu.VMEM((CHUNK, D), jnp.float32),
        pltpu.SemaphoreType.DMA,
        pltpu.SemaphoreType.DMA,
        pltpu.SemaphoreType.DMA,
        pltpu.SemaphoreType.DMA,
        pltpu.SemaphoreType.DMA,
    ],
)
def _mp_sc(h_hbm, row0_hbm, col0_hbm, row1_hbm, col1_hbm, out_hbm, acc,
           col_idx, row_idx, buf_a, buf_b, sem_a, sem_b, sem_sa, sem_sb,
           sem_idx):
    c = lax.axis_index("c")
    s = lax.axis_index("s")
    base = s * ROWS_PER_TILE

    @pl.when(c == 0)
    def _stage0():
        pltpu.async_copy(col0_hbm.at[s, pl.ds(0, IB)], col_idx.at[0],
                         sem_idx)
        pltpu.async_copy(row0_hbm.at[s, pl.ds(0, IB)], row_idx.at[0],
                         sem_idx)

    @pl.when(c == 1)
    def _stage1():
        pltpu.async_copy(col1_hbm.at[s, pl.ds(0, IB)], col_idx.at[0],
                         sem_idx)
        pltpu.async_copy(row1_hbm.at[s, pl.ds(0, IB)], row_idx.at[0],
                         sem_idx)

    z = jnp.zeros((16,), jnp.float32)

    def zrow(r, carry):
        for q in range(D // 16):
            buf_a[r, pl.ds(q * 16, 16)] = z
        return carry

    lax.fori_loop(0, CHUNK, zrow, 0)
    for k in range(ROWS_PER_TILE // CHUNK):
        pltpu.sync_copy(buf_a, acc.at[pl.ds(base + k * CHUNK, CHUNK)])
    rem = ROWS_PER_TILE % CHUNK
    pltpu.sync_copy(buf_a.at[pl.ds(0, rem)],
                    acc.at[pl.ds(base + ROWS_PER_TILE - rem, rem)])

    @pl.when(s == 0)
    def _zero_tail():
        pltpu.sync_copy(buf_a.at[pl.ds(0, TAIL)],
                        acc.at[pl.ds(TAIL_BASE, TAIL)])

    plsc.subcore_barrier()

    def run(rows_hbm, cols_hbm, nchunks):
        ngroups = nchunks // IB
        pltpu.make_async_copy(cols_hbm.at[s, pl.ds(0, IB)],
                              col_idx.at[0], sem_idx).wait()
        pltpu.make_async_copy(rows_hbm.at[s, pl.ds(0, IB)],
                              row_idx.at[0], sem_idx).wait()

        def group(g, gcarry):
            p = lax.rem(g, 2)
            q = 1 - p

            @pl.when(g > 0)
            def _wait_idx():
                pltpu.make_async_copy(cols_hbm.at[s, pl.ds(0, IB)],
                                      col_idx.at[p], sem_idx).wait()
                pltpu.make_async_copy(rows_hbm.at[s, pl.ds(0, IB)],
                                      row_idx.at[p], sem_idx).wait()

            def pair(j, carry):
                @pl.when(jnp.logical_and(j == 1, g + 1 < ngroups))
                def _prefetch():
                    pltpu.async_copy(
                        cols_hbm.at[s, pl.ds((g + 1) * IB, IB)],
                        col_idx.at[q], sem_idx)
                    pltpu.async_copy(
                        rows_hbm.at[s, pl.ds((g + 1) * IB, IB)],
                        row_idx.at[q], sem_idx)
                k0 = 2 * j
                k1 = 2 * j + 1
                not_first = jnp.logical_or(g > 0, j > 0)

                @pl.when(not_first)
                def _wa():
                    pltpu.make_async_copy(
                        h_hbm.at[pl.ds(0, CHUNK)], buf_a, sem_sa).wait()

                cp_a = pltpu.async_copy(h_hbm.at[col_idx.at[p, k0]], buf_a,
                                        sem_a)

                @pl.when(not_first)
                def _wb():
                    pltpu.make_async_copy(
                        h_hbm.at[pl.ds(0, CHUNK)], buf_b, sem_sb).wait()

                cp_b = pltpu.async_copy(h_hbm.at[col_idx.at[p, k1]], buf_b,
                                        sem_b)
                cp_a.wait()
                pltpu.async_copy(buf_a, acc.at[row_idx.at[p, k0]], sem_sa,
                                 add=True)
                cp_b.wait()
                pltpu.async_copy(buf_b, acc.at[row_idx.at[p, k1]], sem_sb,
                                 add=True)
                return carry

            lax.fori_loop(0, IB // 2, pair, 0)
            return gcarry

        lax.fori_loop(0, ngroups, group, 0)
        pltpu.make_async_copy(h_hbm.at[pl.ds(0, CHUNK)], buf_a, sem_sa).wait()
        pltpu.make_async_copy(h_hbm.at[pl.ds(0, CHUNK)], buf_b, sem_sb).wait()

    @pl.when(c == 0)
    def _run0():
        run(row0_hbm, col0_hbm, CH0)

    @pl.when(c == 1)
    def _run1():
        run(row1_hbm, col1_hbm, CH1)

    plsc.subcore_barrier()

    pltpu.sync_copy(acc.at[pl.ds(base, ROWS_PER_TILE)],
                    out_hbm.at[c, pl.ds(base, ROWS_PER_TILE)])

    @pl.when(s == 0)
    def _out_tail():
        pltpu.sync_copy(acc.at[pl.ds(TAIL_BASE, TAIL)],
                        out_hbm.at[c, pl.ds(TAIL_BASE, TAIL)])


_BLK = 2000


def _affine_body(a_ref, h_ref, w_ref, b_ref, o_ref):
    a = a_ref[0] + a_ref[1] + 2.0 * h_ref[...]
    o_ref[...] = jnp.maximum(
        jnp.dot(a, w_ref[...], preferred_element_type=jnp.float32)
        + b_ref[...], 0.0)


def _affine_relu(aggr, h, w, b):
    return pl.pallas_call(
        _affine_body,
        grid=(N // _BLK,),
        in_specs=[
            pl.BlockSpec((NC, _BLK, D), lambda i: (0, i, 0)),
            pl.BlockSpec((_BLK, D), lambda i: (i, 0)),
            pl.BlockSpec((D, D), lambda i: (0, 0)),
            pl.BlockSpec((1, D), lambda i: (0, 0)),
        ],
        out_specs=pl.BlockSpec((_BLK, D), lambda i: (i, 0)),
        out_shape=jax.ShapeDtypeStruct((N, D), jnp.float32),
    )(aggr, h, w, b.reshape(1, D))


def _pool_body(a_ref, h_ref, w_ref, b_ref, batch_ref, wout_ref, bout_ref,
               o_ref, sums_ref, counts_ref):
    i = pl.program_id(0)

    @pl.when(i == 0)
    def _init():
        sums_ref[...] = jnp.zeros_like(sums_ref)
        counts_ref[...] = jnp.zeros_like(counts_ref)

    a = a_ref[0] + a_ref[1] + 2.0 * h_ref[...]
    h = jnp.maximum(
        jnp.dot(a, w_ref[...], preferred_element_type=jnp.float32)
        + b_ref[...], 0.0)
    b = batch_ref[0]
    onehot = (b.reshape(_BLK, 1)
              == lax.broadcasted_iota(jnp.int32, (_BLK, G), 1)
              ).astype(jnp.float32)
    sums_ref[...] += lax.dot_general(
        onehot, h, (((0,), (0,)), ((), ())),
        preferred_element_type=jnp.float32)
    counts_ref[...] += jnp.sum(onehot, axis=0, keepdims=True)

    @pl.when(i == (N // _BLK) - 1)
    def _final():
        pooled = sums_ref[...] / jnp.maximum(counts_ref[...], 1.0).reshape(G, 1)
        o_ref[...] = (jnp.dot(pooled, wout_ref[...],
                              preferred_element_type=jnp.float32)
                      + bout_ref[...])


def _pool_project(aggr, h, w, b, batch3d, wout, bout):
    return pl.pallas_call(
        _pool_body,
        grid=(N // _BLK,),
        in_specs=[
            pl.BlockSpec((NC, _BLK, D), lambda i: (0, i, 0)),
            pl.BlockSpec((_BLK, D), lambda i: (i, 0)),
            pl.BlockSpec((D, D), lambda i: (0, 0)),
            pl.BlockSpec((1, D), lambda i: (0, 0)),
            pl.BlockSpec((1, 1, _BLK), lambda i: (i, 0, 0)),
            pl.BlockSpec((D, D), lambda i: (0, 0)),
            pl.BlockSpec((1, D), lambda i: (0, 0)),
        ],
        out_specs=pl.BlockSpec((G, D), lambda i: (0, 0)),
        out_shape=jax.ShapeDtypeStruct((G, D), jnp.float32),
        scratch_shapes=[
            pltpu.VMEM((G, D), jnp.float32),
            pltpu.VMEM((1, G), jnp.float32),
        ],
    )(aggr, h, w, b.reshape(1, D), batch3d, wout, bout.reshape(1, D))


def kernel(x, edge_index, batch, W1, b1, W2, b2, Wout, bout):
    e = edge_index.shape[1]
    pad = E_PAD - e
    spread = jnp.arange(pad, dtype=jnp.int32)
    row0 = edge_index[0, :E0].reshape(NS, CH0, CHUNK)
    col0 = edge_index[1, :E0].reshape(NS, CH0, CHUNK)
    row1 = jnp.concatenate(
        [edge_index[0, E0:], N + (spread & (CHUNK - 1))]
    ).reshape(NS, CH1, CHUNK)
    col1 = jnp.concatenate(
        [edge_index[1, E0:], spread]).reshape(NS, CH1, CHUNK)
    batch3d = batch.reshape(N // _BLK, 1, _BLK)

    aggr1 = _mp_sc(x, row0, col0, row1, col1)
    h1 = _affine_relu(aggr1, x, W1, b1)
    aggr2 = _mp_sc(h1, row0, col0, row1, col1)
    return _pool_project(aggr2, h1, W2, b2, batch3d, Wout, bout)
